# Initial kernel scaffold; baseline (speedup 1.0000x reference)
import jax
import jax.numpy as jnp
from jax.experimental import pallas as pl


def kernel(hidden_states, top_k_index, top_k_weights, gate_up_proj, down_proj):
    raise NotImplementedError("write your pallas kernel here")


# speedup vs baseline: 3.3613x; 3.3613x over previous
import functools

import jax
import jax.numpy as jnp
from jax import lax
from jax.experimental import pallas as pl
from jax.experimental.pallas import tpu as pltpu
from jax.experimental.pallas import tpu_sc as plsc

NUM_EXPERTS = 8
HIDDEN = 2048
INTER = 1024
TOKENS = 8192
TOPK = 2

BM = 256
NB = (TOKENS * TOPK) // BM + NUM_EXPERTS
C_CAP = NB * BM

NW = 32
ROWS_PER_W = C_CAP // NW
G_CHUNK = 16
N_GCH = ROWS_PER_W // G_CHUNK
T_PER_W = TOKENS // NW
C_CHUNK = 16
N_CCH = T_PER_W // C_CHUNK

_MESH = plsc.VectorSubcoreMesh(core_axis_name="c", subcore_axis_name="s")


@functools.partial(
    pl.kernel,
    out_type=jax.ShapeDtypeStruct((C_CAP, HIDDEN), jnp.float32),
    mesh=_MESH,
    scratch_types=[
        pltpu.VMEM((N_GCH, G_CHUNK), jnp.int32),
        pltpu.VMEM((G_CHUNK, HIDDEN), jnp.float32),
        pltpu.SemaphoreType.DMA,
    ],
)
def _sc_gather(x_hbm, idx_hbm, out_hbm, idx_v, buf, sem):
    wid = lax.axis_index("s") * 2 + lax.axis_index("c")
    pltpu.sync_copy(idx_hbm.at[wid], idx_v)
    base = wid * ROWS_PER_W

    @pl.loop(0, N_GCH)
    def _(ch):
        pltpu.async_copy(x_hbm.at[idx_v.at[ch]], buf, sem).wait()
        pltpu.sync_copy(buf, out_hbm.at[pl.ds(base + ch * G_CHUNK, G_CHUNK)])


@functools.partial(
    pl.kernel,
    out_type=jax.ShapeDtypeStruct((TOKENS, HIDDEN), jnp.float32),
    mesh=_MESH,
    scratch_types=[
        pltpu.VMEM((N_CCH, C_CHUNK), jnp.int32),
        pltpu.VMEM((N_CCH, C_CHUNK), jnp.int32),
        pltpu.VMEM((C_CHUNK, HIDDEN), jnp.float32),
        pltpu.VMEM((C_CHUNK, HIDDEN), jnp.float32),
        pltpu.SemaphoreType.DMA,
        pltpu.SemaphoreType.DMA,
    ],
)
def _sc_combine(dw_hbm, p0_hbm, p1_hbm, out_hbm, p0_v, p1_v, abuf, bbuf, sa, sb):
    wid = lax.axis_index("s") * 2 + lax.axis_index("c")
    pltpu.sync_copy(p0_hbm.at[wid], p0_v)
    pltpu.sync_copy(p1_hbm.at[wid], p1_v)
    base = wid * T_PER_W

    @pl.loop(0, N_CCH)
    def _(ch):
        ca = pltpu.async_copy(dw_hbm.at[p0_v.at[ch]], abuf, sa)
        cb = pltpu.async_copy(dw_hbm.at[p1_v.at[ch]], bbuf, sb)
        ca.wait()
        cb.wait()

        @pl.loop(0, C_CHUNK)
        def _(r):
            @pl.loop(0, HIDDEN, step=64)
            def _(j):
                for u in range(4):
                    slc = (pl.ds(r, 1), pl.ds(j + u * 16, 16))
                    abuf.at[*slc][...] = abuf.at[*slc][...] + bbuf.at[*slc][...]

        pltpu.sync_copy(abuf, out_hbm.at[pl.ds(base + ch * C_CHUNK, C_CHUNK)])


def _mm_body(be_ref, act_ref, x_ref, gu_ref, dn_ref, w_ref, o_ref):
    b = pl.program_id(0)

    @pl.when(act_ref[b] == 1)
    def _():
        xb = x_ref[...].astype(jnp.bfloat16)
        gu = gu_ref[0]
        h = lax.dot_general(xb, gu, (((1,), (1,)), ((), ())),
                            preferred_element_type=jnp.float32)
        gate = h[:, :INTER]
        up = h[:, INTER:]
        act = (gate * jax.nn.sigmoid(gate) * up).astype(jnp.bfloat16)
        d = lax.dot_general(act, dn_ref[0], (((1,), (1,)), ((), ())),
                            preferred_element_type=jnp.float32)
        o_ref[...] = d * w_ref[...]

    @pl.when(act_ref[b] == 0)
    def _():
        o_ref[...] = jnp.zeros_like(o_ref)


def _moe_mm(x_sorted, gu_bf, dn_bf, w_col, be, active):
    grid_spec = pltpu.PrefetchScalarGridSpec(
        num_scalar_prefetch=2,
        grid=(NB,),
        in_specs=[
            pl.BlockSpec((BM, HIDDEN), lambda b, be, ac: (b, 0)),
            pl.BlockSpec((1, 2 * INTER, HIDDEN), lambda b, be, ac: (be[b], 0, 0)),
            pl.BlockSpec((1, HIDDEN, INTER), lambda b, be, ac: (be[b], 0, 0)),
            pl.BlockSpec((BM, 1), lambda b, be, ac: (b, 0)),
        ],
        out_specs=pl.BlockSpec((BM, HIDDEN), lambda b, be, ac: (b, 0)),
    )
    return pl.pallas_call(
        _mm_body,
        grid_spec=grid_spec,
        out_shape=jax.ShapeDtypeStruct((C_CAP, HIDDEN), jnp.float32),
        compiler_params=pltpu.CompilerParams(
            dimension_semantics=("arbitrary",),
            vmem_limit_bytes=60 * 1024 * 1024,
        ),
    )(be, active, x_sorted, gu_bf, dn_bf, w_col)


def kernel(hidden_states, top_k_index, top_k_weights, gate_up_proj, down_proj):
    n = TOKENS * TOPK
    e_flat = top_k_index.reshape(-1).astype(jnp.int32)
    order = jnp.argsort(e_flat, stable=True).astype(jnp.int32)
    e_sorted = e_flat[order]
    counts = jnp.bincount(e_flat, length=NUM_EXPERTS)
    starts = jnp.concatenate(
        [jnp.zeros(1, jnp.int32), jnp.cumsum(counts)[:-1].astype(jnp.int32)])
    padded = ((counts + BM - 1) // BM) * BM
    pcum = jnp.cumsum(padded)
    poff = jnp.concatenate(
        [jnp.zeros(1, jnp.int32), pcum[:-1].astype(jnp.int32)])
    slot = poff[e_sorted] + (jnp.arange(n, dtype=jnp.int32) - starts[e_sorted])
    src_tok = (order // TOPK).astype(jnp.int32)
    gather_idx = jnp.zeros(C_CAP, jnp.int32).at[slot].set(src_tok)
    w_flat = top_k_weights.reshape(-1)
    w_cap = jnp.zeros(C_CAP, jnp.float32).at[slot].set(w_flat[order])
    pos_flat = jnp.zeros(n, jnp.int32).at[order].set(slot)
    block_starts = jnp.arange(NB, dtype=jnp.int32) * BM
    be = jnp.minimum(jnp.searchsorted(pcum, block_starts, side="right"),
                     NUM_EXPERTS - 1).astype(jnp.int32)
    active = (block_starts < pcum[-1]).astype(jnp.int32)

    x_sorted = _sc_gather(hidden_states, gather_idx.reshape(NW, N_GCH, G_CHUNK))

    dw = _moe_mm(
        x_sorted,
        gate_up_proj.astype(jnp.bfloat16),
        down_proj.astype(jnp.bfloat16),
        w_cap[:, None],
        be,
        active,
    )

    p0 = pos_flat[0::TOPK].reshape(NW, N_CCH, C_CHUNK)
    p1 = pos_flat[1::TOPK].reshape(NW, N_CCH, C_CHUNK)
    return _sc_combine(dw, p0, p1)
